# Initial kernel scaffold; baseline (speedup 1.0000x reference)
import jax
import jax.numpy as jnp
from jax.experimental import pallas as pl


def kernel(feat, edge_index_r0, edge_index_r1, edge_index_r2, W1, b1, W2, b2, Wmlp, bmlp):
    raise NotImplementedError("write your pallas kernel here")


# speedup vs baseline: 5.8002x; 5.8002x over previous
import jax
import jax.numpy as jnp
from jax import lax
from jax.experimental import pallas as pl
from jax.experimental.pallas import tpu as pltpu
from jax.experimental.pallas import tpu_sc as plsc

_N = 10000
_E = 320000
_IN = 128
_HID = 64
_NCLS = 2

_NCORES = 2
_NSUB = 16
_NW = _NCORES * _NSUB
_CHUNK = 128
_NCH = 80
_EPW = _CHUNK * _NCH
_EPAD = _EPW * _NW
_NACC = 10240
_RPT = _NACC // _NSUB
_GRP = 4
_NGRP = _NCH // _GRP
_DEGW = 16
_RB = 2000


def _sc_agg(x, srcs, dsts, d, with_deg, zrows_h, onehot_h):
  mesh = plsc.VectorSubcoreMesh(
      core_axis_name="c", subcore_axis_name="s",
      num_cores=_NCORES, num_subcores=_NSUB)
  out_type = [jax.ShapeDtypeStruct((_NCORES, 3, _NACC, d), jnp.float32)]
  if with_deg:
    out_type.append(
        jax.ShapeDtypeStruct((_NCORES, _NACC, _DEGW), jnp.float32))
  scratch = [
      pltpu.VMEM((_NCH, _CHUNK), jnp.int32),
      pltpu.VMEM((_NCH, _CHUNK), jnp.int32),
      pltpu.VMEM((_GRP, _CHUNK, d), jnp.float32),
      pltpu.VMEM_SHARED((_NACC, d), jnp.float32),
      pltpu.SemaphoreType.DMA,
      pltpu.SemaphoreType.DMA,
  ]
  if with_deg:
    scratch += [
        pltpu.VMEM((_CHUNK, _DEGW), jnp.float32),
        pltpu.VMEM_SHARED((_NACC, _DEGW), jnp.float32),
    ]

  def body(x_hbm, s0, d0, s1, d1, s2, d2, *rest):
    if with_deg:
      (zr_hbm, zd_hbm, oh_hbm, out_hbm, deg_hbm, srcv, dstv, rows, acc,
       gsem, ssem, onesv, dacc) = rest
    else:
      (zr_hbm, out_hbm, srcv, dstv, rows, acc, gsem, ssem) = rest
    c = lax.axis_index("c")
    s = lax.axis_index("s")
    w = c * _NSUB + s
    rbase = s * _RPT

    pltpu.sync_copy(zr_hbm, acc.at[pl.ds(rbase, _RPT)])
    if with_deg:
      pltpu.sync_copy(zd_hbm, dacc.at[pl.ds(rbase, _RPT)])
    plsc.subcore_barrier()

    for r, (src_h, dst_h) in enumerate(((s0, d0), (s1, d1), (s2, d2))):
      pltpu.sync_copy(src_h.at[pl.ds(w * _NCH, _NCH)], srcv)
      pltpu.sync_copy(dst_h.at[pl.ds(w * _NCH, _NCH)], dstv)
      if with_deg:
        pltpu.sync_copy(oh_hbm.at[r], onesv)

      @pl.loop(0, _NGRP)
      def _(g):
        gh = []
        for b in range(_GRP):
          gh.append(pltpu.async_copy(
              x_hbm.at[srcv.at[g * _GRP + b]], rows.at[b], gsem))
        for h in gh:
          h.wait()
        sh = []
        for b in range(_GRP):
          sh.append(pltpu.async_copy(
              rows.at[b], acc.at[dstv.at[g * _GRP + b]], ssem, add=True))
          if with_deg:
            sh.append(pltpu.async_copy(
                onesv, dacc.at[dstv.at[g * _GRP + b]], ssem, add=True))
        for h in sh:
          h.wait()

      plsc.subcore_barrier()
      pltpu.sync_copy(acc.at[pl.ds(rbase, _RPT)],
                      out_hbm.at[c, r, pl.ds(rbase, _RPT)])
      if r < 2:
        pltpu.sync_copy(zr_hbm, acc.at[pl.ds(rbase, _RPT)])
      plsc.subcore_barrier()

    if with_deg:
      pltpu.sync_copy(dacc.at[pl.ds(rbase, _RPT)],
                      deg_hbm.at[c, pl.ds(rbase, _RPT)])

  k = pl.kernel(body, out_type=out_type, mesh=mesh, scratch_types=scratch,
                compiler_params=pltpu.CompilerParams(
                    use_tc_tiling_on_sc=False))
  if with_deg:
    zd_h = jnp.zeros((_RPT, _DEGW), jnp.float32)
    return k(x, srcs[0], dsts[0], srcs[1], dsts[1], srcs[2], dsts[2],
             zrows_h, zd_h, onehot_h)
  return k(x, srcs[0], dsts[0], srcs[1], dsts[1], srcs[2], dsts[2], zrows_h)


def _tc1(feat, W1, Wmlp, bmlp2d):
  def body(f_ref, w1_ref, wm_ref, bm_ref, g1_ref, sim_ref):
    f = f_ref[...]
    g1_ref[...] = jnp.dot(f, w1_ref[...], preferred_element_type=jnp.float32)
    sim_ref[...] = jnp.tanh(
        jnp.dot(f, wm_ref[...], preferred_element_type=jnp.float32)
        + bm_ref[...])

  grid = _N // _RB
  return pl.pallas_call(
      body,
      grid=(grid,),
      in_specs=[
          pl.BlockSpec((_RB, _IN), lambda i: (i, 0)),
          pl.BlockSpec((_IN, _HID), lambda i: (0, 0)),
          pl.BlockSpec((_IN, _NCLS), lambda i: (0, 0)),
          pl.BlockSpec((1, _NCLS), lambda i: (0, 0)),
      ],
      out_specs=[
          pl.BlockSpec((_RB, _HID), lambda i: (i, 0)),
          pl.BlockSpec((_RB, _NCLS), lambda i: (i, 0)),
      ],
      out_shape=[
          jax.ShapeDtypeStruct((_N, _HID), jnp.float32),
          jax.ShapeDtypeStruct((_N, _NCLS), jnp.float32),
      ],
  )(feat, W1, Wmlp, bmlp2d)


def _tc2(S1, DG, g1, b1_2d, W2p):
  def body(s_ref, dg_ref, g1_ref, b1_ref, w2_ref, g2_ref):
    ssum = s_ref[0] + s_ref[1]
    dg = dg_ref[0] + dg_ref[1]
    inv = 1.0 / jnp.maximum(dg, 1.0)
    p = (ssum[0] * inv[:, 0:1] + ssum[1] * inv[:, 1:2]
         + ssum[2] * inv[:, 2:3])
    h1 = 0.5 * p + g1_ref[...] + b1_ref[...]
    g2_ref[...] = jnp.dot(h1, w2_ref[...],
                          preferred_element_type=jnp.float32)

  grid = _N // _RB
  return pl.pallas_call(
      body,
      grid=(grid,),
      in_specs=[
          pl.BlockSpec((2, 3, _RB, _HID), lambda i: (0, 0, i, 0)),
          pl.BlockSpec((2, _RB, _DEGW), lambda i: (0, i, 0)),
          pl.BlockSpec((_RB, _HID), lambda i: (i, 0)),
          pl.BlockSpec((1, _HID), lambda i: (0, 0)),
          pl.BlockSpec((_HID, _DEGW), lambda i: (0, 0)),
      ],
      out_specs=pl.BlockSpec((_RB, _DEGW), lambda i: (i, 0)),
      out_shape=jax.ShapeDtypeStruct((_N, _DEGW), jnp.float32),
  )(S1, DG, g1, b1_2d, W2p)


def _tc3(S2, DG, g2, b2_2d):
  def body(s_ref, dg_ref, g2_ref, b2_ref, o_ref):
    ssum = s_ref[0] + s_ref[1]
    dg = dg_ref[0] + dg_ref[1]
    inv = 1.0 / jnp.maximum(dg, 1.0)
    o = 0.5 * (ssum[0] * inv[:, 0:1] + ssum[1] * inv[:, 1:2]
               + ssum[2] * inv[:, 2:3]) + g2_ref[...]
    o_ref[...] = o[:, 0:_NCLS] + b2_ref[...]

  grid = _N // _RB
  return pl.pallas_call(
      body,
      grid=(grid,),
      in_specs=[
          pl.BlockSpec((2, 3, _RB, _DEGW), lambda i: (0, 0, i, 0)),
          pl.BlockSpec((2, _RB, _DEGW), lambda i: (0, i, 0)),
          pl.BlockSpec((_RB, _DEGW), lambda i: (i, 0)),
          pl.BlockSpec((1, _NCLS), lambda i: (0, 0)),
      ],
      out_specs=pl.BlockSpec((_RB, _NCLS), lambda i: (i, 0)),
      out_shape=jax.ShapeDtypeStruct((_N, _NCLS), jnp.float32),
  )(S2, DG, g2, b2_2d)


def kernel(feat, edge_index_r0, edge_index_r1, edge_index_r2,
           W1, b1, W2, b2, Wmlp, bmlp):
  pad = _EPAD - _E
  srcs, dsts = [], []
  for ei in (edge_index_r0, edge_index_r1, edge_index_r2):
    srcs.append(jnp.concatenate(
        [ei[0], jnp.zeros((pad,), jnp.int32)]).reshape(_EPAD // _CHUNK,
                                                       _CHUNK))
    dsts.append(jnp.concatenate(
        [ei[1], jnp.full((pad,), _N, jnp.int32)]).reshape(_EPAD // _CHUNK,
                                                          _CHUNK))

  g1, sim = _tc1(feat, W1, Wmlp, bmlp.reshape(1, _NCLS))

  onehot = jnp.broadcast_to(
      jnp.eye(3, _DEGW, dtype=jnp.float32)[:, None, :],
      (3, _CHUNK, _DEGW))
  S1, DEG = _sc_agg(g1, srcs, dsts, _HID, True,
                    jnp.zeros((_RPT, _HID), jnp.float32), onehot)
  S1 = S1[:, :, :_N]
  DEG = DEG[:, :_N]

  W2p = jnp.pad(W2, ((0, 0), (0, _DEGW - _NCLS)))
  g2 = _tc2(S1, DEG, g1, b1.reshape(1, _HID), W2p)

  S2 = _sc_agg(g2, srcs, dsts, _DEGW, False,
               jnp.zeros((_RPT, _DEGW), jnp.float32), None)[0][:, :, :_N]
  out = _tc3(S2, DEG, g2, b2.reshape(1, _NCLS))
  return (out, sim)
